# baseline (speedup 1.0000x reference)
import functools

import jax
import jax.numpy as jnp
from jax import lax
from jax.experimental import pallas as pl
from jax.experimental.pallas import tpu as pltpu
from jax.experimental.pallas import tpu_sc as plsc

B = 4096
NS = 26
ND = 13
V = 100000
D = 16
L = 20
NSEQ = 2

NW = 32
BPW = B // NW
E2_ROWS = BPW * NS
SEQ_ROWS = BPW * L


def _sc_gather(idx_sp, idx_sa, idx_sg, E1, E2, Eseq):
    mesh = plsc.VectorSubcoreMesh(core_axis_name="c", subcore_axis_name="s")

    @functools.partial(
        pl.kernel,
        out_type=[
            jax.ShapeDtypeStruct((B, NS * D), jnp.float32),
            jax.ShapeDtypeStruct((B, NSEQ * D), jnp.float32),
            jax.ShapeDtypeStruct((B,), jnp.float32),
        ],
        mesh=mesh,
        compiler_params=pltpu.CompilerParams(use_tc_tiling_on_sc=False),
        scratch_types=[
            pltpu.VMEM((NS * BPW,), jnp.int32),
            pltpu.VMEM((E2_ROWS, D), jnp.float32),
            pltpu.VMEM((SEQ_ROWS,), jnp.int32),
            pltpu.VMEM((SEQ_ROWS, D), jnp.float32),
            pltpu.VMEM((E2_ROWS,), jnp.float32),
            pltpu.VMEM((BPW, NSEQ * D), jnp.float32),
            pltpu.VMEM((BPW,), jnp.float32),
            pltpu.SemaphoreType.DMA,
            pltpu.SemaphoreType.DMA,
            pltpu.SemaphoreType.DMA,
            pltpu.SemaphoreType.DMA,
        ],
    )
    def k(idx_sp_h, idx_sa_h, idx_sg_h, E1_h, E2_h, Eseq_h,
          emb_out, seq_out, lin_out,
          idx2_v, rows2_v, idxsa_v, rowss_v, e1_v, pooled_v, lin_v,
          sem_bulk, sem_sa, sem_sg, sem_wr):
        wid = lax.axis_index("s") * 2 + lax.axis_index("c")
        bbase = wid * BPW

        pltpu.sync_copy(idx_sp_h.at[pl.ds(bbase * NS, NS * BPW)], idx2_v)
        sbase = wid * SEQ_ROWS
        pltpu.sync_copy(idx_sa_h.at[pl.ds(sbase, SEQ_ROWS)], idxsa_v)

        d_sa = pltpu.async_copy(Eseq_h.at[0].at[idxsa_v], rowss_v, sem_sa)
        d_bulk = [
            pltpu.async_copy(E2_h.at[0].at[idx2_v], rows2_v, sem_bulk),
            pltpu.async_copy(E1_h.at[0].at[idx2_v], e1_v, sem_bulk),
        ]

        d_sa.wait()

        def pool_a(bl, _):
            acc = jnp.zeros((D,), jnp.float32)
            for l in range(L):
                acc = acc + rowss_v[bl * L + l, :]
            pooled_v[bl, pl.ds(0, D)] = acc * (1.0 / L)
            return 0

        lax.fori_loop(0, BPW, pool_a, 0)
        pltpu.sync_copy(idx_sg_h.at[pl.ds(sbase, SEQ_ROWS)], idxsa_v)
        d_sg = pltpu.async_copy(Eseq_h.at[1].at[idxsa_v], rowss_v, sem_sg)

        for d in d_bulk:
            d.wait()
        d_wr = []
        for i in range(NS):
            d_wr.append(pltpu.async_copy(
                rows2_v.at[pl.ds(i * BPW, BPW)],
                emb_out.at[pl.ds(bbase, BPW), pl.ds(i * D, D)], sem_wr))

        def lin_body(c, _):
            acc = jnp.zeros((D,), jnp.float32)
            for i in range(NS):
                acc = acc + e1_v[pl.ds(i * BPW + c * D, D)]
            lin_v[pl.ds(c * D, D)] = acc
            return 0

        lax.fori_loop(0, BPW // D, lin_body, 0)
        pltpu.sync_copy(lin_v, lin_out.at[pl.ds(bbase, BPW)])

        d_sg.wait()

        def pool_g(bl, _):
            acc = jnp.zeros((D,), jnp.float32)
            for l in range(L):
                acc = acc + rowss_v[bl * L + l, :]
            pooled_v[bl, pl.ds(D, D)] = acc * (1.0 / L)
            return 0

        lax.fori_loop(0, BPW, pool_g, 0)
        pltpu.sync_copy(pooled_v, seq_out.at[pl.ds(bbase, BPW)])
        for d in d_wr:
            d.wait()

    return k(idx_sp, idx_sa, idx_sg, E1, E2, Eseq)


_TC_BLK = 512


def _tc_body(dense_r, emb_r, seqp_r, lin_r, W1d_r, W1e_r, W1s_r, b1_r,
             W2_r, b2_r, W3_r, b3_r, W4_r, b4_r, Wlin_r, blin_r,
             Wf_r, bf_r, Wl_r, bl_r, S26_r, S2_r, fin_o, like_o):
    f32 = jnp.float32
    dot = lambda a, b: lax.dot(a, b, preferred_element_type=f32)
    xd = dense_r[...]
    xe = emb_r[...]
    xs = seqp_r[...]
    h = dot(xd, W1d_r[...]) + dot(xe, W1e_r[...]) + dot(xs, W1s_r[...]) + b1_r[...]
    h = jnp.maximum(h, 0.0)
    h = jnp.maximum(dot(h, W2_r[...]) + b2_r[...], 0.0)
    h = jnp.maximum(dot(h, W3_r[...]) + b3_r[...], 0.0)
    dnn = dot(h, W4_r[...]) + b4_r[...]
    summed = dot(xe, S26_r[...]) + dot(xs, S2_r[...])
    sqsum = jnp.sum(xe * xe, axis=1, keepdims=True)
    so = 0.5 * (jnp.sum(summed * summed, axis=1, keepdims=True) - sqsum)
    fo = dot(xd, Wlin_r[...]) + blin_r[...] + lin_r[...]
    logits = fo + so + dnn
    fin_o[...] = jax.nn.sigmoid(logits * Wf_r[0, 0] + bf_r[0, 0])
    like_o[...] = jax.nn.sigmoid(logits * Wl_r[0, 0] + bl_r[0, 0])


def _tc_head(dense, emb, seqp, lin, W1d, W1e, W1s, b1, W2, b2, W3, b3,
             W4, b4, Wlin, blin, Wf, bf, Wl, bl, S26, S2):
    n_blk = B // _TC_BLK

    def bspec(shape):
        return pl.BlockSpec(shape, lambda i: tuple(0 for _ in shape))

    in_specs = [
        pl.BlockSpec((_TC_BLK, ND), lambda i: (i, 0)),
        pl.BlockSpec((_TC_BLK, NS * D), lambda i: (i, 0)),
        pl.BlockSpec((_TC_BLK, NSEQ * D), lambda i: (i, 0)),
        pl.BlockSpec((_TC_BLK, 1), lambda i: (i, 0)),
        bspec(W1d.shape), bspec(W1e.shape), bspec(W1s.shape), bspec(b1.shape),
        bspec(W2.shape), bspec(b2.shape), bspec(W3.shape), bspec(b3.shape),
        bspec(W4.shape), bspec(b4.shape), bspec(Wlin.shape), bspec(blin.shape),
        bspec(Wf.shape), bspec(bf.shape), bspec(Wl.shape), bspec(bl.shape),
        bspec(S26.shape), bspec(S2.shape),
    ]
    out_specs = [
        pl.BlockSpec((_TC_BLK, 1), lambda i: (i, 0)),
        pl.BlockSpec((_TC_BLK, 1), lambda i: (i, 0)),
    ]
    return pl.pallas_call(
        _tc_body,
        grid=(n_blk,),
        in_specs=in_specs,
        out_specs=out_specs,
        out_shape=[
            jax.ShapeDtypeStruct((B, 1), jnp.float32),
            jax.ShapeDtypeStruct((B, 1), jnp.float32),
        ],
    )(dense, emb, seqp, lin, W1d, W1e, W1s, b1, W2, b2, W3, b3, W4, b4,
      Wlin, blin, Wf, bf, Wl, bl, S26, S2)


def kernel(sparse_inputs, dense_inputs, seq_actors, seq_genres, E1, E2, Eseq,
           Wlin, blin, W1, b1, W2, b2, W3, b3, W4, b4, Wf, bf, Wl, bl):
    si = sparse_inputs.astype(jnp.int32)
    offs = jnp.arange(NS, dtype=jnp.int32) * V
    idx_sp = (si + offs[None, :]).reshape(NW, BPW, NS).transpose(0, 2, 1).reshape(-1)
    idx_sa = seq_actors.astype(jnp.int32).reshape(-1)
    idx_sg = seq_genres.astype(jnp.int32).reshape(-1)

    E1sq = E1.reshape(NS, V)
    emb, seqp, lin_sum = _sc_gather(idx_sp, idx_sa, idx_sg, E1sq, E2, Eseq)
    lin = lin_sum.reshape(B, 1)

    S26 = jnp.tile(jnp.eye(D, dtype=jnp.float32), (NS, 1))
    S2 = jnp.tile(jnp.eye(D, dtype=jnp.float32), (NSEQ, 1))
    W1d = W1[:ND]
    W1e = W1[ND:ND + NS * D]
    W1s = W1[ND + NS * D:]

    fin, like = _tc_head(
        dense_inputs, emb, seqp, lin, W1d, W1e, W1s, b1.reshape(1, -1),
        W2, b2.reshape(1, -1), W3, b3.reshape(1, -1), W4, b4.reshape(1, -1),
        Wlin, blin.reshape(1, 1), Wf, bf.reshape(1, 1), Wl, bl.reshape(1, 1),
        S26, S2)
    return (fin, like)


# speedup vs baseline: 2.0201x; 2.0201x over previous
import functools

import jax
import jax.numpy as jnp
from jax import lax
from jax.experimental import pallas as pl
from jax.experimental.pallas import tpu as pltpu
from jax.experimental.pallas import tpu_sc as plsc

B = 4096
NS = 26
ND = 13
V = 100000
D = 16
L = 20
NSEQ = 2

NW = 32
BPW = B // NW
E2_ROWS = BPW * NS
SEQ_ROWS = BPW * L


def _sc_gather_e2(idx_all, E2t):
    mesh = plsc.VectorSubcoreMesh(core_axis_name="c", subcore_axis_name="s")

    @functools.partial(
        pl.kernel,
        out_type=jax.ShapeDtypeStruct((D, B * NS), jnp.float32),
        mesh=mesh,
        compiler_params=pltpu.CompilerParams(use_tc_tiling_on_sc=False),
        scratch_types=[
            pltpu.VMEM((D * E2_ROWS,), jnp.int32),
            pltpu.VMEM((D * E2_ROWS,), jnp.float32),
            pltpu.SemaphoreType.DMA,
            pltpu.SemaphoreType.DMA,
        ],
    )
    def k(idx_all_h, E2t_h, out_dm, idx_v, val_v, sem_g, sem_wr):
        wid = lax.axis_index("s") * 2 + lax.axis_index("c")
        rbase = wid * E2_ROWS

        for d in range(D):
            pltpu.sync_copy(idx_all_h.at[d, pl.ds(rbase, E2_ROWS)],
                            idx_v.at[pl.ds(d * E2_ROWS, E2_ROWS)])

        e2_flat = E2t_h.at[0].at[0]
        ds_g = []
        for d in range(D):
            ds_g.append(pltpu.async_copy(
                e2_flat.at[idx_v.at[pl.ds(d * E2_ROWS, E2_ROWS)]],
                val_v.at[pl.ds(d * E2_ROWS, E2_ROWS)], sem_g))
        d_wr = []
        for d in range(D):
            ds_g[d].wait()
            d_wr.append(pltpu.async_copy(
                val_v.at[pl.ds(d * E2_ROWS, E2_ROWS)],
                out_dm.at[d, pl.ds(rbase, E2_ROWS)], sem_wr))
        for d in d_wr:
            d.wait()

    return k(idx_all, E2t)


def _sc_gather_rest(idx_sp, idx_sa, idx_sg, E1sq, Eseq):
    mesh = plsc.VectorSubcoreMesh(core_axis_name="c", subcore_axis_name="s")

    @functools.partial(
        pl.kernel,
        out_type=[
            jax.ShapeDtypeStruct((B, NSEQ * D), jnp.float32),
            jax.ShapeDtypeStruct((B,), jnp.float32),
        ],
        mesh=mesh,
        compiler_params=pltpu.CompilerParams(use_tc_tiling_on_sc=False),
        scratch_types=[
            pltpu.VMEM((E2_ROWS,), jnp.int32),
            pltpu.VMEM((SEQ_ROWS,), jnp.int32),
            pltpu.VMEM((SEQ_ROWS, D), jnp.float32),
            pltpu.VMEM((E2_ROWS,), jnp.float32),
            pltpu.VMEM((BPW, NSEQ * D), jnp.float32),
            pltpu.VMEM((BPW,), jnp.float32),
            pltpu.SemaphoreType.DMA,
            pltpu.SemaphoreType.DMA,
            pltpu.SemaphoreType.DMA,
        ],
    )
    def k(idx_sp_h, idx_sa_h, idx_sg_h, E1_h, Eseq_h,
          seq_out, lin_out,
          idx2_v, idxsa_v, rowss_v, e1_v, pooled_v, lin_v,
          sem_e1, sem_sa, sem_sg):
        wid = lax.axis_index("s") * 2 + lax.axis_index("c")
        bbase = wid * BPW
        sbase = wid * SEQ_ROWS

        pltpu.sync_copy(idx_sp_h.at[pl.ds(bbase * NS, E2_ROWS)], idx2_v)
        pltpu.sync_copy(idx_sa_h.at[pl.ds(sbase, SEQ_ROWS)], idxsa_v)

        d_sa = pltpu.async_copy(Eseq_h.at[0].at[idxsa_v], rowss_v, sem_sa)
        d_e1 = pltpu.async_copy(E1_h.at[0].at[idx2_v], e1_v, sem_e1)

        d_sa.wait()

        def pool_a(bl, _):
            acc = jnp.zeros((D,), jnp.float32)
            for l in range(L):
                acc = acc + rowss_v[bl * L + l, :]
            pooled_v[bl, pl.ds(0, D)] = acc * (1.0 / L)
            return 0

        lax.fori_loop(0, BPW, pool_a, 0)
        pltpu.sync_copy(idx_sg_h.at[pl.ds(sbase, SEQ_ROWS)], idxsa_v)
        d_sg = pltpu.async_copy(Eseq_h.at[1].at[idxsa_v], rowss_v, sem_sg)

        d_e1.wait()

        def lin_body(c, _):
            acc = jnp.zeros((D,), jnp.float32)
            for i in range(NS):
                acc = acc + e1_v[pl.ds(i * BPW + c * D, D)]
            lin_v[pl.ds(c * D, D)] = acc
            return 0

        lax.fori_loop(0, BPW // D, lin_body, 0)
        pltpu.sync_copy(lin_v, lin_out.at[pl.ds(bbase, BPW)])

        d_sg.wait()

        def pool_g(bl, _):
            acc = jnp.zeros((D,), jnp.float32)
            for l in range(L):
                acc = acc + rowss_v[bl * L + l, :]
            pooled_v[bl, pl.ds(D, D)] = acc * (1.0 / L)
            return 0

        lax.fori_loop(0, BPW, pool_g, 0)
        pltpu.sync_copy(pooled_v, seq_out.at[pl.ds(bbase, BPW)])

    return k(idx_sp, idx_sa, idx_sg, E1sq, Eseq)


_TC_BLK = 512


def _tc_body(dense_r, emb_r, seqp_r, lin_r, W1d_r, W1e_r, W1s_r, b1_r,
             W2_r, b2_r, W3_r, b3_r, W4_r, b4_r, Wlin_r, blin_r,
             Wf_r, bf_r, Wl_r, bl_r, ones26_r, fin_o, like_o):
    f32 = jnp.float32
    dot = lambda a, b: lax.dot(a, b, preferred_element_type=f32)
    xd = dense_r[...]
    xs = seqp_r[...]
    h = dot(xd, W1d_r[...]) + dot(xs, W1s_r[...]) + b1_r[...]
    ones26 = ones26_r[...]
    sq = jnp.zeros((_TC_BLK, 1), f32)
    sqsum = jnp.zeros((_TC_BLK, 1), f32)
    for d in range(D):
        Ed = emb_r[d]
        h = h + dot(Ed, W1e_r[d])
        sd = jnp.sum(Ed, axis=1, keepdims=True)
        tot = sd + xs[:, d:d + 1] + xs[:, D + d:D + d + 1]
        sq = sq + tot * tot
        sqsum = sqsum + jnp.sum(Ed * Ed, axis=1, keepdims=True)
    h = jnp.maximum(h, 0.0)
    h = jnp.maximum(dot(h, W2_r[...]) + b2_r[...], 0.0)
    h = jnp.maximum(dot(h, W3_r[...]) + b3_r[...], 0.0)
    dnn = dot(h, W4_r[...]) + b4_r[...]
    so = 0.5 * (sq - sqsum)
    fo = dot(xd, Wlin_r[...]) + blin_r[...] + lin_r[...]
    logits = fo + so + dnn
    fin_o[...] = jax.nn.sigmoid(logits * Wf_r[0, 0] + bf_r[0, 0])
    like_o[...] = jax.nn.sigmoid(logits * Wl_r[0, 0] + bl_r[0, 0])


def _tc_head(dense, emb3, seqp, lin, W1d, W1e_dm, W1s, b1, W2, b2, W3, b3,
             W4, b4, Wlin, blin, Wf, bf, Wl, bl, ones26):
    n_blk = B // _TC_BLK

    def bspec(shape):
        return pl.BlockSpec(shape, lambda i: tuple(0 for _ in shape))

    in_specs = [
        pl.BlockSpec((_TC_BLK, ND), lambda i: (i, 0)),
        pl.BlockSpec((D, _TC_BLK, NS), lambda i: (0, i, 0)),
        pl.BlockSpec((_TC_BLK, NSEQ * D), lambda i: (i, 0)),
        pl.BlockSpec((_TC_BLK, 1), lambda i: (i, 0)),
        bspec(W1d.shape), bspec(W1e_dm.shape), bspec(W1s.shape), bspec(b1.shape),
        bspec(W2.shape), bspec(b2.shape), bspec(W3.shape), bspec(b3.shape),
        bspec(W4.shape), bspec(b4.shape), bspec(Wlin.shape), bspec(blin.shape),
        bspec(Wf.shape), bspec(bf.shape), bspec(Wl.shape), bspec(bl.shape),
        bspec(ones26.shape),
    ]
    out_specs = [
        pl.BlockSpec((_TC_BLK, 1), lambda i: (i, 0)),
        pl.BlockSpec((_TC_BLK, 1), lambda i: (i, 0)),
    ]
    return pl.pallas_call(
        _tc_body,
        grid=(n_blk,),
        in_specs=in_specs,
        out_specs=out_specs,
        out_shape=[
            jax.ShapeDtypeStruct((B, 1), jnp.float32),
            jax.ShapeDtypeStruct((B, 1), jnp.float32),
        ],
    )(dense, emb3, seqp, lin, W1d, W1e_dm, W1s, b1, W2, b2, W3, b3, W4, b4,
      Wlin, blin, Wf, bf, Wl, bl, ones26)


def kernel(sparse_inputs, dense_inputs, seq_actors, seq_genres, E1, E2, Eseq,
           Wlin, blin, W1, b1, W2, b2, W3, b3, W4, b4, Wf, bf, Wl, bl):
    si = sparse_inputs.astype(jnp.int32)
    offs = jnp.arange(NS, dtype=jnp.int32) * (D * V)
    base = si + offs[None, :]
    doff = jnp.arange(D, dtype=jnp.int32) * V
    idx_all = (doff[:, None, None] + base[None, :, :]).reshape(D, B * NS)
    E2t = E2.transpose(0, 2, 1)

    offs1 = jnp.arange(NS, dtype=jnp.int32) * V
    idx_sp = (si + offs1[None, :]).reshape(NW, BPW, NS).transpose(0, 2, 1).reshape(-1)
    idx_sa = seq_actors.astype(jnp.int32).reshape(-1)
    idx_sg = seq_genres.astype(jnp.int32).reshape(-1)
    E1sq = E1.reshape(NS, V)

    emb_dm = _sc_gather_e2(idx_all, E2t)
    seqp, lin_sum = _sc_gather_rest(idx_sp, idx_sa, idx_sg, E1sq, Eseq)
    emb3 = emb_dm.reshape(D, B, NS)
    lin = lin_sum.reshape(B, 1)

    W1e_dm = W1[ND:ND + NS * D].reshape(NS, D, 200).transpose(1, 0, 2)
    ones26 = jnp.ones((NS, 1), jnp.float32)
    W1d = W1[:ND]
    W1s = W1[ND + NS * D:]

    fin, like = _tc_head(
        dense_inputs, emb3, seqp, lin, W1d, W1e_dm, W1s, b1.reshape(1, -1),
        W2, b2.reshape(1, -1), W3, b3.reshape(1, -1), W4, b4.reshape(1, -1),
        Wlin, blin.reshape(1, 1), Wf, bf.reshape(1, 1), Wl, bl.reshape(1, 1),
        ones26)
    return (fin, like)


# speedup vs baseline: 2.0868x; 1.0331x over previous
import functools

import jax
import jax.numpy as jnp
from jax import lax
from jax.experimental import pallas as pl
from jax.experimental.pallas import tpu as pltpu
from jax.experimental.pallas import tpu_sc as plsc

B = 4096
NS = 26
ND = 13
V = 100000
D = 16
L = 20
NSEQ = 2

NW = 32
BPW = B // NW
E2_ROWS = BPW * NS
SEQ_ROWS = BPW * L


def _sc_gather_e2(idx_all, E2t):
    mesh = plsc.VectorSubcoreMesh(core_axis_name="c", subcore_axis_name="s")

    @functools.partial(
        pl.kernel,
        out_type=jax.ShapeDtypeStruct((D, B * NS), jnp.float32),
        mesh=mesh,
        compiler_params=pltpu.CompilerParams(use_tc_tiling_on_sc=False),
        scratch_types=[
            pltpu.VMEM((D * E2_ROWS,), jnp.int32),
            pltpu.VMEM((D * E2_ROWS,), jnp.float32),
            pltpu.SemaphoreType.DMA,
            pltpu.SemaphoreType.DMA,
        ],
    )
    def k(idx_all_h, E2t_h, out_dm, idx_v, val_v, sem_g, sem_wr):
        wid = lax.axis_index("s") * 2 + lax.axis_index("c")
        rbase = wid * E2_ROWS

        for d in range(D):
            pltpu.sync_copy(idx_all_h.at[d, pl.ds(rbase, E2_ROWS)],
                            idx_v.at[pl.ds(d * E2_ROWS, E2_ROWS)])

        e2_flat = E2t_h.at[0].at[0]
        ds_g = []
        for d in range(D):
            ds_g.append(pltpu.async_copy(
                e2_flat.at[idx_v.at[pl.ds(d * E2_ROWS, E2_ROWS)]],
                val_v.at[pl.ds(d * E2_ROWS, E2_ROWS)], sem_g))
        d_wr = []
        for d in range(D):
            ds_g[d].wait()
            d_wr.append(pltpu.async_copy(
                val_v.at[pl.ds(d * E2_ROWS, E2_ROWS)],
                out_dm.at[d, pl.ds(rbase, E2_ROWS)], sem_wr))
        for d in d_wr:
            d.wait()

    return k(idx_all, E2t)


def _sc_gather_rest(idx_sp, idx_sa, idx_sg, E1sq, Eseq):
    mesh = plsc.VectorSubcoreMesh(core_axis_name="c", subcore_axis_name="s")

    @functools.partial(
        pl.kernel,
        out_type=[
            jax.ShapeDtypeStruct((B, NSEQ * D), jnp.float32),
            jax.ShapeDtypeStruct((B,), jnp.float32),
        ],
        mesh=mesh,
        compiler_params=pltpu.CompilerParams(use_tc_tiling_on_sc=False),
        scratch_types=[
            pltpu.VMEM((E2_ROWS,), jnp.int32),
            pltpu.VMEM((SEQ_ROWS,), jnp.int32),
            pltpu.VMEM((SEQ_ROWS, D), jnp.float32),
            pltpu.VMEM((E2_ROWS,), jnp.float32),
            pltpu.VMEM((BPW, NSEQ * D), jnp.float32),
            pltpu.VMEM((BPW,), jnp.float32),
            pltpu.SemaphoreType.DMA,
            pltpu.SemaphoreType.DMA,
            pltpu.SemaphoreType.DMA,
        ],
    )
    def k(idx_sp_h, idx_sa_h, idx_sg_h, E1_h, Eseq_h,
          seq_out, lin_out,
          idx2_v, idxsa_v, rowss_v, e1_v, pooled_v, lin_v,
          sem_e1, sem_sa, sem_sg):
        wid = lax.axis_index("s") * 2 + lax.axis_index("c")
        bbase = wid * BPW
        sbase = wid * SEQ_ROWS

        pltpu.sync_copy(idx_sp_h.at[pl.ds(bbase * NS, E2_ROWS)], idx2_v)
        pltpu.sync_copy(idx_sa_h.at[pl.ds(sbase, SEQ_ROWS)], idxsa_v)

        d_sa = pltpu.async_copy(Eseq_h.at[0].at[idxsa_v], rowss_v, sem_sa)
        d_e1 = pltpu.async_copy(E1_h.at[0].at[idx2_v], e1_v, sem_e1)

        d_sa.wait()

        def pool_a(bl, _):
            acc = jnp.zeros((D,), jnp.float32)
            for l in range(L):
                acc = acc + rowss_v[bl * L + l, :]
            pooled_v[bl, pl.ds(0, D)] = acc * (1.0 / L)
            return 0

        lax.fori_loop(0, BPW, pool_a, 0)
        pltpu.sync_copy(idx_sg_h.at[pl.ds(sbase, SEQ_ROWS)], idxsa_v)
        d_sg = pltpu.async_copy(Eseq_h.at[1].at[idxsa_v], rowss_v, sem_sg)

        d_e1.wait()

        def lin_body(c, _):
            acc = jnp.zeros((D,), jnp.float32)
            for i in range(NS):
                acc = acc + e1_v[pl.ds(i * BPW + c * D, D)]
            lin_v[pl.ds(c * D, D)] = acc
            return 0

        lax.fori_loop(0, BPW // D, lin_body, 0)
        pltpu.sync_copy(lin_v, lin_out.at[pl.ds(bbase, BPW)])

        d_sg.wait()

        def pool_g(bl, _):
            acc = jnp.zeros((D,), jnp.float32)
            for l in range(L):
                acc = acc + rowss_v[bl * L + l, :]
            pooled_v[bl, pl.ds(D, D)] = acc * (1.0 / L)
            return 0

        lax.fori_loop(0, BPW, pool_g, 0)
        pltpu.sync_copy(pooled_v, seq_out.at[pl.ds(bbase, BPW)])

    return k(idx_sp, idx_sa, idx_sg, E1sq, Eseq)


_TC_BLK = 512


def _tc_body(dense_r, emb_r, seqp_r, lin_r, W1d_r, W1e_r, W1s_r, b1_r,
             W2_r, b2_r, W3_r, b3_r, W4_r, b4_r, Wlin_r, blin_r,
             Wf_r, bf_r, Wl_r, bl_r, ones26_r, fin_o, like_o):
    f32 = jnp.float32
    dot = lambda a, b: lax.dot(a, b, preferred_element_type=f32)
    xd = dense_r[...]
    xs = seqp_r[...]
    h = dot(xd, W1d_r[...]) + dot(xs, W1s_r[...]) + b1_r[...]
    ones26 = ones26_r[...]
    E = emb_r[...].reshape(D, _TC_BLK, NS)
    sq = jnp.zeros((_TC_BLK, 1), f32)
    sqsum = jnp.zeros((_TC_BLK, 1), f32)
    for d in range(D):
        Ed = E[d]
        h = h + dot(Ed, W1e_r[d])
        sd = jnp.sum(Ed, axis=1, keepdims=True)
        tot = sd + xs[:, d:d + 1] + xs[:, D + d:D + d + 1]
        sq = sq + tot * tot
        sqsum = sqsum + jnp.sum(Ed * Ed, axis=1, keepdims=True)
    h = jnp.maximum(h, 0.0)
    h = jnp.maximum(dot(h, W2_r[...]) + b2_r[...], 0.0)
    h = jnp.maximum(dot(h, W3_r[...]) + b3_r[...], 0.0)
    dnn = dot(h, W4_r[...]) + b4_r[...]
    so = 0.5 * (sq - sqsum)
    fo = dot(xd, Wlin_r[...]) + blin_r[...] + lin_r[...]
    logits = fo + so + dnn
    fin_o[...] = jax.nn.sigmoid(logits * Wf_r[0, 0] + bf_r[0, 0])
    like_o[...] = jax.nn.sigmoid(logits * Wl_r[0, 0] + bl_r[0, 0])


def _tc_head(dense, emb3, seqp, lin, W1d, W1e_dm, W1s, b1, W2, b2, W3, b3,
             W4, b4, Wlin, blin, Wf, bf, Wl, bl, ones26):
    n_blk = B // _TC_BLK

    def bspec(shape):
        return pl.BlockSpec(shape, lambda i: tuple(0 for _ in shape))

    in_specs = [
        pl.BlockSpec((_TC_BLK, ND), lambda i: (i, 0)),
        pl.BlockSpec((D, _TC_BLK * NS), lambda i: (0, i)),
        pl.BlockSpec((_TC_BLK, NSEQ * D), lambda i: (i, 0)),
        pl.BlockSpec((_TC_BLK, 1), lambda i: (i, 0)),
        bspec(W1d.shape), bspec(W1e_dm.shape), bspec(W1s.shape), bspec(b1.shape),
        bspec(W2.shape), bspec(b2.shape), bspec(W3.shape), bspec(b3.shape),
        bspec(W4.shape), bspec(b4.shape), bspec(Wlin.shape), bspec(blin.shape),
        bspec(Wf.shape), bspec(bf.shape), bspec(Wl.shape), bspec(bl.shape),
        bspec(ones26.shape),
    ]
    out_specs = [
        pl.BlockSpec((_TC_BLK, 1), lambda i: (i, 0)),
        pl.BlockSpec((_TC_BLK, 1), lambda i: (i, 0)),
    ]
    return pl.pallas_call(
        _tc_body,
        grid=(n_blk,),
        in_specs=in_specs,
        out_specs=out_specs,
        out_shape=[
            jax.ShapeDtypeStruct((B, 1), jnp.float32),
            jax.ShapeDtypeStruct((B, 1), jnp.float32),
        ],
    )(dense, emb3, seqp, lin, W1d, W1e_dm, W1s, b1, W2, b2, W3, b3, W4, b4,
      Wlin, blin, Wf, bf, Wl, bl, ones26)


def kernel(sparse_inputs, dense_inputs, seq_actors, seq_genres, E1, E2, Eseq,
           Wlin, blin, W1, b1, W2, b2, W3, b3, W4, b4, Wf, bf, Wl, bl):
    si = sparse_inputs.astype(jnp.int32)
    offs = jnp.arange(NS, dtype=jnp.int32) * (D * V)
    base = si + offs[None, :]
    doff = jnp.arange(D, dtype=jnp.int32) * V
    idx_all = (doff[:, None, None] + base[None, :, :]).reshape(D, B * NS)
    E2t = E2.transpose(0, 2, 1)

    offs1 = jnp.arange(NS, dtype=jnp.int32) * V
    idx_sp = (si + offs1[None, :]).reshape(NW, BPW, NS).transpose(0, 2, 1).reshape(-1)
    idx_sa = seq_actors.astype(jnp.int32).reshape(-1)
    idx_sg = seq_genres.astype(jnp.int32).reshape(-1)
    E1sq = E1.reshape(NS, V)

    emb3 = _sc_gather_e2(idx_all, E2t)
    seqp, lin_sum = _sc_gather_rest(idx_sp, idx_sa, idx_sg, E1sq, Eseq)
    lin = lin_sum.reshape(B, 1)

    W1e_dm = W1[ND:ND + NS * D].reshape(NS, D, 200).transpose(1, 0, 2)
    ones26 = jnp.ones((NS, 1), jnp.float32)
    W1d = W1[:ND]
    W1s = W1[ND + NS * D:]

    fin, like = _tc_head(
        dense_inputs, emb3, seqp, lin, W1d, W1e_dm, W1s, b1.reshape(1, -1),
        W2, b2.reshape(1, -1), W3, b3.reshape(1, -1), W4, b4.reshape(1, -1),
        Wlin, blin.reshape(1, 1), Wf, bf.reshape(1, 1), Wl, bl.reshape(1, 1),
        ones26)
    return (fin, like)


# speedup vs baseline: 2.1377x; 1.0244x over previous
import functools

import jax
import jax.numpy as jnp
from jax import lax
from jax.experimental import pallas as pl
from jax.experimental.pallas import tpu as pltpu
from jax.experimental.pallas import tpu_sc as plsc

B = 4096
NS = 26
ND = 13
V = 100000
D = 16
L = 20
NSEQ = 2

NW = 32
BPW = B // NW
E2_ROWS = BPW * NS
SEQ_ROWS = BPW * L


def _sc_gather_e2(idx_all, E2t):
    mesh = plsc.VectorSubcoreMesh(core_axis_name="c", subcore_axis_name="s")

    @functools.partial(
        pl.kernel,
        out_type=jax.ShapeDtypeStruct((D, B * NS), jnp.float32),
        mesh=mesh,
        compiler_params=pltpu.CompilerParams(use_tc_tiling_on_sc=False),
        scratch_types=[
            pltpu.VMEM((D * E2_ROWS,), jnp.int32),
            pltpu.VMEM((D * E2_ROWS,), jnp.float32),
            pltpu.SemaphoreType.DMA,
            pltpu.SemaphoreType.DMA,
        ],
    )
    def k(idx_all_h, E2t_h, out_dm, idx_v, val_v, sem_g, sem_wr):
        wid = lax.axis_index("s") * 2 + lax.axis_index("c")
        rbase = wid * E2_ROWS

        for d in range(D):
            pltpu.sync_copy(idx_all_h.at[d, pl.ds(rbase, E2_ROWS)],
                            idx_v.at[pl.ds(d * E2_ROWS, E2_ROWS)])

        e2_flat = E2t_h.at[0].at[0]
        ds_g = []
        for d in range(D):
            ds_g.append(pltpu.async_copy(
                e2_flat.at[idx_v.at[pl.ds(d * E2_ROWS, E2_ROWS)]],
                val_v.at[pl.ds(d * E2_ROWS, E2_ROWS)], sem_g))
        d_wr = []
        for d in range(D):
            ds_g[d].wait()
            d_wr.append(pltpu.async_copy(
                val_v.at[pl.ds(d * E2_ROWS, E2_ROWS)],
                out_dm.at[d, pl.ds(rbase, E2_ROWS)], sem_wr))
        for d in d_wr:
            d.wait()

    return k(idx_all, E2t)


def _sc_gather_rest(idx_sp, idx_sa, idx_sg, E1t, Eseq):
    mesh = plsc.VectorSubcoreMesh(core_axis_name="c", subcore_axis_name="s")

    @functools.partial(
        pl.kernel,
        out_type=[
            jax.ShapeDtypeStruct((B, NSEQ * D), jnp.float32),
            jax.ShapeDtypeStruct((B,), jnp.float32),
        ],
        mesh=mesh,
        compiler_params=pltpu.CompilerParams(use_tc_tiling_on_sc=False),
        scratch_types=[
            pltpu.VMEM((E2_ROWS,), jnp.int32),
            pltpu.VMEM((SEQ_ROWS,), jnp.int32),
            pltpu.VMEM((SEQ_ROWS, D), jnp.float32),
            pltpu.VMEM((E2_ROWS,), jnp.float32),
            pltpu.VMEM((BPW, NSEQ * D), jnp.float32),
            pltpu.VMEM((BPW,), jnp.float32),
            pltpu.SemaphoreType.DMA,
            pltpu.SemaphoreType.DMA,
            pltpu.SemaphoreType.DMA,
        ],
    )
    def k(idx_sp_h, idx_sa_h, idx_sg_h, E1_h, Eseq_h,
          seq_out, lin_out,
          idx2_v, idxsa_v, rowss_v, e1_v, pooled_v, lin_v,
          sem_e1, sem_sa, sem_sg):
        wid = lax.axis_index("s") * 2 + lax.axis_index("c")
        bbase = wid * BPW
        sbase = wid * SEQ_ROWS

        pltpu.sync_copy(idx_sp_h.at[pl.ds(bbase * NS, E2_ROWS)], idx2_v)
        pltpu.sync_copy(idx_sa_h.at[pl.ds(sbase, SEQ_ROWS)], idxsa_v)

        d_sa = pltpu.async_copy(Eseq_h.at[0].at[idxsa_v], rowss_v, sem_sa)
        d_e1 = pltpu.async_copy(E1_h.at[0].at[0].at[idx2_v], e1_v, sem_e1)

        d_sa.wait()

        def pool_a(bl, _):
            acc = jnp.zeros((D,), jnp.float32)
            for l in range(L):
                acc = acc + rowss_v[bl * L + l, :]
            pooled_v[bl, pl.ds(0, D)] = acc * (1.0 / L)
            return 0

        lax.fori_loop(0, BPW, pool_a, 0)
        pltpu.sync_copy(idx_sg_h.at[pl.ds(sbase, SEQ_ROWS)], idxsa_v)
        d_sg = pltpu.async_copy(Eseq_h.at[1].at[idxsa_v], rowss_v, sem_sg)

        d_e1.wait()

        def lin_body(c, _):
            acc = jnp.zeros((D,), jnp.float32)
            for i in range(NS):
                acc = acc + e1_v[pl.ds(i * BPW + c * D, D)]
            lin_v[pl.ds(c * D, D)] = acc
            return 0

        lax.fori_loop(0, BPW // D, lin_body, 0)
        pltpu.sync_copy(lin_v, lin_out.at[pl.ds(bbase, BPW)])

        d_sg.wait()

        def pool_g(bl, _):
            acc = jnp.zeros((D,), jnp.float32)
            for l in range(L):
                acc = acc + rowss_v[bl * L + l, :]
            pooled_v[bl, pl.ds(D, D)] = acc * (1.0 / L)
            return 0

        lax.fori_loop(0, BPW, pool_g, 0)
        pltpu.sync_copy(pooled_v, seq_out.at[pl.ds(bbase, BPW)])

    return k(idx_sp, idx_sa, idx_sg, E1t, Eseq)


_TC_BLK = 512


def _tc_body(dense_r, emb_r, seqp_r, lin_r, W1d_r, W1e_r, W1s_r, b1_r,
             W2_r, b2_r, W3_r, b3_r, W4_r, b4_r, Wlin_r, blin_r,
             Wf_r, bf_r, Wl_r, bl_r, ones26_r, fin_o, like_o):
    f32 = jnp.float32
    dot = lambda a, b: lax.dot(a, b, preferred_element_type=f32)
    xd = dense_r[...]
    xs = seqp_r[...]
    h = dot(xd, W1d_r[...]) + dot(xs, W1s_r[...]) + b1_r[...]
    ones26 = ones26_r[...]
    E = emb_r[...].reshape(D, _TC_BLK, NS)
    sq = jnp.zeros((_TC_BLK, 1), f32)
    sqsum = jnp.zeros((_TC_BLK, 1), f32)
    for d in range(D):
        Ed = E[d]
        h = h + dot(Ed, W1e_r[d])
        sd = jnp.sum(Ed, axis=1, keepdims=True)
        tot = sd + xs[:, d:d + 1] + xs[:, D + d:D + d + 1]
        sq = sq + tot * tot
        sqsum = sqsum + jnp.sum(Ed * Ed, axis=1, keepdims=True)
    h = jnp.maximum(h, 0.0)
    h = jnp.maximum(dot(h, W2_r[...]) + b2_r[...], 0.0)
    h = jnp.maximum(dot(h, W3_r[...]) + b3_r[...], 0.0)
    dnn = dot(h, W4_r[...]) + b4_r[...]
    so = 0.5 * (sq - sqsum)
    fo = dot(xd, Wlin_r[...]) + blin_r[...] + lin_r[...]
    logits = fo + so + dnn
    fin_o[...] = jax.nn.sigmoid(logits * Wf_r[0, 0] + bf_r[0, 0])
    like_o[...] = jax.nn.sigmoid(logits * Wl_r[0, 0] + bl_r[0, 0])


def _tc_head(dense, emb3, seqp, lin, W1d, W1e_dm, W1s, b1, W2, b2, W3, b3,
             W4, b4, Wlin, blin, Wf, bf, Wl, bl, ones26):
    n_blk = B // _TC_BLK

    def bspec(shape):
        return pl.BlockSpec(shape, lambda i: tuple(0 for _ in shape))

    in_specs = [
        pl.BlockSpec((_TC_BLK, ND), lambda i: (i, 0)),
        pl.BlockSpec((D, _TC_BLK * NS), lambda i: (0, i)),
        pl.BlockSpec((_TC_BLK, NSEQ * D), lambda i: (i, 0)),
        pl.BlockSpec((_TC_BLK, 1), lambda i: (i, 0)),
        bspec(W1d.shape), bspec(W1e_dm.shape), bspec(W1s.shape), bspec(b1.shape),
        bspec(W2.shape), bspec(b2.shape), bspec(W3.shape), bspec(b3.shape),
        bspec(W4.shape), bspec(b4.shape), bspec(Wlin.shape), bspec(blin.shape),
        bspec(Wf.shape), bspec(bf.shape), bspec(Wl.shape), bspec(bl.shape),
        bspec(ones26.shape),
    ]
    out_specs = [
        pl.BlockSpec((_TC_BLK, 1), lambda i: (i, 0)),
        pl.BlockSpec((_TC_BLK, 1), lambda i: (i, 0)),
    ]
    return pl.pallas_call(
        _tc_body,
        grid=(n_blk,),
        in_specs=in_specs,
        out_specs=out_specs,
        out_shape=[
            jax.ShapeDtypeStruct((B, 1), jnp.float32),
            jax.ShapeDtypeStruct((B, 1), jnp.float32),
        ],
    )(dense, emb3, seqp, lin, W1d, W1e_dm, W1s, b1, W2, b2, W3, b3, W4, b4,
      Wlin, blin, Wf, bf, Wl, bl, ones26)


def kernel(sparse_inputs, dense_inputs, seq_actors, seq_genres, E1, E2, Eseq,
           Wlin, blin, W1, b1, W2, b2, W3, b3, W4, b4, Wf, bf, Wl, bl):
    si = sparse_inputs.astype(jnp.int32)
    offs = jnp.arange(NS, dtype=jnp.int32) * (D * V)
    base = si + offs[None, :]
    doff = jnp.arange(D, dtype=jnp.int32) * V
    idx_all = (doff[:, None, None] + base[None, :, :]).reshape(D, B * NS)
    E2t = E2.transpose(0, 2, 1)

    offs1 = jnp.arange(NS, dtype=jnp.int32) * V
    idx_sp = (si + offs1[None, :]).reshape(NW, BPW, NS).transpose(0, 2, 1).reshape(-1)
    idx_sa = seq_actors.astype(jnp.int32).reshape(-1)
    idx_sg = seq_genres.astype(jnp.int32).reshape(-1)
    E1t = E1.transpose(0, 2, 1)

    emb3 = _sc_gather_e2(idx_all, E2t)
    seqp, lin_sum = _sc_gather_rest(idx_sp, idx_sa, idx_sg, E1t, Eseq)
    lin = lin_sum.reshape(B, 1)

    W1e_dm = W1[ND:ND + NS * D].reshape(NS, D, 200).transpose(1, 0, 2)
    ones26 = jnp.ones((NS, 1), jnp.float32)
    W1d = W1[:ND]
    W1s = W1[ND + NS * D:]

    fin, like = _tc_head(
        dense_inputs, emb3, seqp, lin, W1d, W1e_dm, W1s, b1.reshape(1, -1),
        W2, b2.reshape(1, -1), W3, b3.reshape(1, -1), W4, b4.reshape(1, -1),
        Wlin, blin.reshape(1, 1), Wf, bf.reshape(1, 1), Wl, bl.reshape(1, 1),
        ones26)
    return (fin, like)
